# baseline (speedup 1.0000x reference)
import functools

import jax
import jax.numpy as jnp
from jax.experimental import pallas as pl
from jax.experimental.pallas import tpu as pltpu

_N = 10000
_M = 5000
_K = 32
_D = 128
_SUB = 8
_LANES = 1280
_NPAD = _SUB * _LANES
_IDXL = 640
_BC = 200
_RB = 512


def _fps_body(pr_ref, px_ref, py_ref, pz_ref, idx_ref):
    si = jax.lax.broadcasted_iota(jnp.int32, (_SUB, _LANES), 0)
    li = jax.lax.broadcasted_iota(jnp.int32, (_SUB, _LANES), 1)
    fj = si * _LANES + li
    d0 = jnp.where(fj < _N, jnp.inf, -jnp.inf)
    px = px_ref[...]
    py = py_ref[...]
    pz = pz_ref[...]

    def body(i, carry):
        dists, cur = carry
        idx_ref[i] = cur
        c = pr_ref[pl.ds(cur, 1), :]
        dx = px - c[0:1, 0:1]
        dy = py - c[0:1, 1:2]
        dz = pz - c[0:1, 2:3]
        d = dx * dx + dy * dy + dz * dz
        nd = jnp.minimum(dists, d)
        mx = jnp.max(nd)
        nxt = jnp.min(jnp.where(nd == mx, fj, _NPAD)).astype(jnp.int32)
        return nd, nxt

    jax.lax.fori_loop(0, _M, body, (d0, jnp.int32(0)))


_KROWS = 80
_CPP = 2


def _knn_body(ps_ref, px_ref, py_ref, pz_ref, nbr_ref):
    ri = jax.lax.broadcasted_iota(jnp.int32, (_KROWS, 128), 0)
    ci = jax.lax.broadcasted_iota(jnp.int32, (_KROWS, 128), 1)
    jiota = ri * 128 + ci
    px = px_ref[...]
    py = py_ref[...]
    pz = pz_ref[...]
    d2s = []
    for c in range(_CPP):
        cx = ps_ref[0, c:c + 1, 0:1]
        cy = ps_ref[0, c:c + 1, 1:2]
        cz = ps_ref[0, c:c + 1, 2:3]
        dx = cx - px
        dy = cy - py
        dz = cz - pz
        d2s.append(dx * dx + dy * dy + dz * dz)
    js = [[] for _ in range(_CPP)]
    for _ in range(_K):
        for c in range(_CPP):
            m = jnp.min(d2s[c])
            j = jnp.min(jnp.where(d2s[c] == m, jiota, _NPAD))
            d2s[c] = jnp.where(jiota == j, jnp.inf, d2s[c])
            js[c].append(j)
    for c in range(_CPP):
        nbr_ref[0, c:c + 1, :] = jnp.stack(js[c])[None, :]


def _u_body(x_ref, p_ref, w1a_ref, w1b_ref, b1_ref, u_ref):
    u_ref[...] = (
        jnp.dot(x_ref[...], w1a_ref[...], preferred_element_type=jnp.float32)
        + jnp.dot(p_ref[...], w1b_ref[...], preferred_element_type=jnp.float32)
        + b1_ref[...]
    )


def _mlp_body(g_ref, ps_ref, w1b_ref, w2_ref, b2_ref, out_ref):
    v = jnp.dot(ps_ref[...], w1b_ref[...], preferred_element_type=jnp.float32)
    acc = jnp.full((_BC, _D), -jnp.inf, dtype=jnp.float32)
    w2 = w2_ref[...]
    b2 = b2_ref[...]
    for k in range(_K):
        h1 = jnp.maximum(g_ref[k] - v, 0.0)
        h2 = jnp.dot(h1, w2, preferred_element_type=jnp.float32) + b2
        acc = jnp.maximum(acc, h2)
    out_ref[...] = jnp.maximum(acc, 0.0)


def kernel(x, pos, batch, W1, b1, W2, b2):
    f32 = jnp.float32
    posp = jnp.pad(pos.astype(f32), ((0, _NPAD - _N), (0, 0)))
    px = posp[:, 0].reshape(_SUB, _LANES)
    py = posp[:, 1].reshape(_SUB, _LANES)
    pz = posp[:, 2].reshape(_SUB, _LANES)

    idx_buf = pl.pallas_call(
        _fps_body,
        in_specs=[
            pl.BlockSpec((_NPAD, 3), lambda: (0, 0)),
            pl.BlockSpec((_SUB, _LANES), lambda: (0, 0)),
            pl.BlockSpec((_SUB, _LANES), lambda: (0, 0)),
            pl.BlockSpec((_SUB, _LANES), lambda: (0, 0)),
        ],
        out_specs=pl.BlockSpec(memory_space=pltpu.SMEM),
        out_shape=jax.ShapeDtypeStruct((_SUB * _IDXL,), jnp.int32),
    )(posp, px, py, pz)
    idx = idx_buf[:_M]

    pos_s = jnp.take(pos, idx, axis=0)

    posq = jnp.pad(pos.astype(f32), ((0, _NPAD - _N), (0, 0)),
                   constant_values=1e4)
    pxq = posq[:, 0].reshape(_KROWS, 128)
    pyq = posq[:, 1].reshape(_KROWS, 128)
    pzq = posq[:, 2].reshape(_KROWS, 128)
    ps3 = pos_s.reshape(_M // _CPP, _CPP, 3)
    nbr3 = pl.pallas_call(
        _knn_body,
        grid=(_M // _CPP,),
        in_specs=[
            pl.BlockSpec((1, _CPP, 3), lambda b: (b, 0, 0)),
            pl.BlockSpec((_KROWS, 128), lambda b: (0, 0)),
            pl.BlockSpec((_KROWS, 128), lambda b: (0, 0)),
            pl.BlockSpec((_KROWS, 128), lambda b: (0, 0)),
        ],
        out_specs=pl.BlockSpec((1, _CPP, _K), lambda b: (b, 0, 0)),
        out_shape=jax.ShapeDtypeStruct((_M // _CPP, _CPP, _K), jnp.int32),
    )(ps3, pxq, pyq, pzq)
    nbr = nbr3.reshape(_M, _K)

    W1a = W1[:_D, :]
    W1b = W1[_D:, :]
    b1r = b1.reshape(1, _D)
    xp = jnp.pad(x.astype(f32), ((0, _NPAD - _N), (0, 0)))
    u = pl.pallas_call(
        _u_body,
        grid=(_NPAD // _RB,),
        in_specs=[
            pl.BlockSpec((_RB, _D), lambda b: (b, 0)),
            pl.BlockSpec((_RB, 3), lambda b: (b, 0)),
            pl.BlockSpec((_D, _D), lambda b: (0, 0)),
            pl.BlockSpec((3, _D), lambda b: (0, 0)),
            pl.BlockSpec((1, _D), lambda b: (0, 0)),
        ],
        out_specs=pl.BlockSpec((_RB, _D), lambda b: (b, 0)),
        out_shape=jax.ShapeDtypeStruct((_NPAD, _D), f32),
    )(xp, posp, W1a, W1b, b1r)

    col = nbr.T.reshape(-1)
    g = jnp.take(u, col, axis=0).reshape(_K, _M, _D)

    b2r = b2.reshape(1, _D)
    out = pl.pallas_call(
        _mlp_body,
        grid=(_M // _BC,),
        in_specs=[
            pl.BlockSpec((_K, _BC, _D), lambda b: (0, b, 0)),
            pl.BlockSpec((_BC, 3), lambda b: (b, 0)),
            pl.BlockSpec((3, _D), lambda b: (0, 0)),
            pl.BlockSpec((_D, _D), lambda b: (0, 0)),
            pl.BlockSpec((1, _D), lambda b: (0, 0)),
        ],
        out_specs=pl.BlockSpec((_BC, _D), lambda b: (b, 0)),
        out_shape=jax.ShapeDtypeStruct((_M, _D), f32),
    )(g, pos_s, W1b, W2, b2r)

    return (out, pos_s, jnp.take(batch, idx, axis=0))


# speedup vs baseline: 2.9363x; 2.9363x over previous
import functools

import jax
import jax.numpy as jnp
from jax.experimental import pallas as pl
from jax.experimental.pallas import tpu as pltpu

_N = 10000
_M = 5000
_K = 32
_D = 128
_SUB = 8
_LANES = 1280
_NPAD = _SUB * _LANES
_IDXL = 640
_BC = 200
_RB = 512


def _fps_body(pr_ref, px_ref, py_ref, pz_ref, idx_ref):
    si = jax.lax.broadcasted_iota(jnp.int32, (_SUB, _LANES), 0)
    li = jax.lax.broadcasted_iota(jnp.int32, (_SUB, _LANES), 1)
    fj = si * _LANES + li
    d0 = jnp.where(fj < _N, jnp.inf, -jnp.inf)
    px = px_ref[...]
    py = py_ref[...]
    pz = pz_ref[...]

    def body(i, carry):
        dists, cur = carry
        idx_ref[i] = cur
        c = pr_ref[pl.ds(cur, 1), :]
        dx = px - c[0:1, 0:1]
        dy = py - c[0:1, 1:2]
        dz = pz - c[0:1, 2:3]
        d = dx * dx + dy * dy + dz * dz
        nd = jnp.minimum(dists, d)
        rm = jnp.max(nd, axis=1, keepdims=True)
        ra = jnp.min(jnp.where(nd == rm, fj, _NPAD), axis=1,
                     keepdims=True)
        gm = jnp.max(rm)
        nxt = jnp.min(jnp.where(rm == gm, ra, _NPAD)).astype(jnp.int32)
        return nd, nxt

    jax.lax.fori_loop(0, _M, body, (d0, jnp.int32(0)))


_NCH = 8
_CW = _NPAD // _NCH


def _knn_body(ps_ref, px_ref, py_ref, pz_ref, nbr_ref):
    ps = ps_ref[...]
    cx = ps[:, 0:1]
    cy = ps[:, 1:2]
    cz = ps[:, 2:3]
    li = jax.lax.broadcasted_iota(jnp.int32, (_SUB, _CW), 1)
    cand_v = []
    cand_i = []
    for g in range(_NCH):
        sl = slice(g * _CW, (g + 1) * _CW)
        dx = cx - px_ref[:, sl]
        dy = cy - py_ref[:, sl]
        dz = cz - pz_ref[:, sl]
        d2 = dx * dx + dy * dy + dz * dz
        gi = li + (g * _CW)
        for k in range(_K):
            m = jnp.min(d2, axis=1, keepdims=True)
            am = jnp.min(jnp.where(d2 == m, gi, _NPAD), axis=1, keepdims=True)
            cand_v.append(m)
            cand_i.append(am)
            d2 = jnp.where(gi == am, jnp.inf, d2)
    vals = jnp.concatenate(cand_v, axis=1)
    idxs = jnp.concatenate(cand_i, axis=1)
    for k in range(_K):
        m = jnp.min(vals, axis=1, keepdims=True)
        j = jnp.min(jnp.where(vals == m, idxs, _NPAD), axis=1, keepdims=True)
        nbr_ref[:, k:k + 1] = j
        sel = jnp.logical_and(vals == m, idxs == j)
        vals = jnp.where(sel, jnp.inf, vals)


def _u_body(x_ref, p_ref, w1a_ref, w1b_ref, b1_ref, u_ref):
    u_ref[...] = (
        jnp.dot(x_ref[...], w1a_ref[...], preferred_element_type=jnp.float32)
        + jnp.dot(p_ref[...], w1b_ref[...], preferred_element_type=jnp.float32)
        + b1_ref[...]
    )


def _mlp_body(g_ref, ps_ref, w1b_ref, w2_ref, b2_ref, out_ref):
    v = jnp.dot(ps_ref[...], w1b_ref[...], preferred_element_type=jnp.float32)
    acc = jnp.full((_BC, _D), -jnp.inf, dtype=jnp.float32)
    w2 = w2_ref[...]
    b2 = b2_ref[...]
    for k in range(_K):
        h1 = jnp.maximum(g_ref[k] - v, 0.0)
        h2 = jnp.dot(h1, w2, preferred_element_type=jnp.float32) + b2
        acc = jnp.maximum(acc, h2)
    out_ref[...] = jnp.maximum(acc, 0.0)


def kernel(x, pos, batch, W1, b1, W2, b2):
    f32 = jnp.float32
    posp = jnp.pad(pos.astype(f32), ((0, _NPAD - _N), (0, 0)))
    px = posp[:, 0].reshape(_SUB, _LANES)
    py = posp[:, 1].reshape(_SUB, _LANES)
    pz = posp[:, 2].reshape(_SUB, _LANES)

    idx_buf = pl.pallas_call(
        _fps_body,
        in_specs=[
            pl.BlockSpec((_NPAD, 3), lambda: (0, 0)),
            pl.BlockSpec((_SUB, _LANES), lambda: (0, 0)),
            pl.BlockSpec((_SUB, _LANES), lambda: (0, 0)),
            pl.BlockSpec((_SUB, _LANES), lambda: (0, 0)),
        ],
        out_specs=pl.BlockSpec(memory_space=pltpu.SMEM),
        out_shape=jax.ShapeDtypeStruct((_SUB * _IDXL,), jnp.int32),
    )(posp, px, py, pz)
    idx = idx_buf[:_M]

    pos_s = jnp.take(pos, idx, axis=0)

    posq = jnp.pad(pos.astype(f32), ((0, _NPAD - _N), (0, 0)),
                   constant_values=1e4)
    pxr = posq[:, 0].reshape(1, _NPAD)
    pyr = posq[:, 1].reshape(1, _NPAD)
    pzr = posq[:, 2].reshape(1, _NPAD)
    nbr = pl.pallas_call(
        _knn_body,
        grid=(_M // _SUB,),
        in_specs=[
            pl.BlockSpec((_SUB, 3), lambda b: (b, 0)),
            pl.BlockSpec((1, _NPAD), lambda b: (0, 0)),
            pl.BlockSpec((1, _NPAD), lambda b: (0, 0)),
            pl.BlockSpec((1, _NPAD), lambda b: (0, 0)),
        ],
        out_specs=pl.BlockSpec((_SUB, _K), lambda b: (b, 0)),
        out_shape=jax.ShapeDtypeStruct((_M, _K), jnp.int32),
    )(pos_s, pxr, pyr, pzr)

    W1a = W1[:_D, :]
    W1b = W1[_D:, :]
    b1r = b1.reshape(1, _D)
    xp = jnp.pad(x.astype(f32), ((0, _NPAD - _N), (0, 0)))
    u = pl.pallas_call(
        _u_body,
        grid=(_NPAD // _RB,),
        in_specs=[
            pl.BlockSpec((_RB, _D), lambda b: (b, 0)),
            pl.BlockSpec((_RB, 3), lambda b: (b, 0)),
            pl.BlockSpec((_D, _D), lambda b: (0, 0)),
            pl.BlockSpec((3, _D), lambda b: (0, 0)),
            pl.BlockSpec((1, _D), lambda b: (0, 0)),
        ],
        out_specs=pl.BlockSpec((_RB, _D), lambda b: (b, 0)),
        out_shape=jax.ShapeDtypeStruct((_NPAD, _D), f32),
    )(xp, posp, W1a, W1b, b1r)

    col = nbr.T.reshape(-1)
    g = jnp.take(u, col, axis=0).reshape(_K, _M, _D)

    b2r = b2.reshape(1, _D)
    out = pl.pallas_call(
        _mlp_body,
        grid=(_M // _BC,),
        in_specs=[
            pl.BlockSpec((_K, _BC, _D), lambda b: (0, b, 0)),
            pl.BlockSpec((_BC, 3), lambda b: (b, 0)),
            pl.BlockSpec((3, _D), lambda b: (0, 0)),
            pl.BlockSpec((_D, _D), lambda b: (0, 0)),
            pl.BlockSpec((1, _D), lambda b: (0, 0)),
        ],
        out_specs=pl.BlockSpec((_BC, _D), lambda b: (b, 0)),
        out_shape=jax.ShapeDtypeStruct((_M, _D), f32),
    )(g, pos_s, W1b, W2, b2r)

    return (out, pos_s, jnp.take(batch, idx, axis=0))


# speedup vs baseline: 4.1279x; 1.4058x over previous
import functools

import jax
import jax.numpy as jnp
from jax import lax
from jax.experimental import pallas as pl
from jax.experimental.pallas import tpu as pltpu
from jax.experimental.pallas import tpu_sc as plsc

_N = 10000
_M = 5000
_K = 32
_D = 128
_SUB = 8
_LANES = 1280
_NPAD = _SUB * _LANES
_IDXL = 640
_BC = 200
_RB = 512


def _fps_body(pr_ref, px_ref, py_ref, pz_ref, idx_ref):
    si = jax.lax.broadcasted_iota(jnp.int32, (_SUB, _LANES), 0)
    li = jax.lax.broadcasted_iota(jnp.int32, (_SUB, _LANES), 1)
    fj = si * _LANES + li
    d0 = jnp.where(fj < _N, jnp.inf, -jnp.inf)

    def body(i, carry):
        dists, cur = carry
        idx_ref[i] = cur
        c = pr_ref[pl.ds(cur, 1), :]
        dx = px_ref[...] - c[0:1, 0:1]
        d = dx * dx
        dy = py_ref[...] - c[0:1, 1:2]
        d = d + dy * dy
        dz = pz_ref[...] - c[0:1, 2:3]
        d = d + dz * dz
        nd = jnp.minimum(dists, d)
        rm = jnp.max(nd, axis=1, keepdims=True)
        la = jnp.min(jnp.where(nd == rm,
                               jax.lax.broadcasted_iota(
                                   jnp.int32, (_SUB, _LANES), 1),
                               _LANES), axis=1, keepdims=True)
        roff = jax.lax.broadcasted_iota(jnp.int32, (_SUB, 1), 0) * _LANES
        ra = la + roff
        gm = jnp.max(rm)
        nxt = jnp.min(jnp.where(rm == gm, ra, _NPAD)).astype(jnp.int32)
        return nd, nxt

    jax.lax.fori_loop(0, _M, body, (d0, jnp.int32(0)))


_NCH = 32
_CW = _NPAD // _NCH


_CG = 8


def _knn_body(ps_ref, px_ref, py_ref, pz_ref, nbr_ref, sv_ref, si_ref):
    ps = ps_ref[...]
    cx = ps[:, 0:1]
    cy = ps[:, 1:2]
    cz = ps[:, 2:3]
    li = jax.lax.broadcasted_iota(jnp.int32, (_SUB, _CW), 1).astype(jnp.float32)
    for grp in range(_NCH // _CG):
        d2s = []
        for t in range(_CG):
            g = grp * _CG + t
            sl = slice(g * _CW, (g + 1) * _CW)
            dx = cx - px_ref[:, sl]
            dy = cy - py_ref[:, sl]
            dz = cz - pz_ref[:, sl]
            d2s.append(dx * dx + dy * dy + dz * dz)
        for k in range(_K):
            for t in range(_CG):
                g = grp * _CG + t
                m = jnp.min(d2s[t], axis=1, keepdims=True)
                aml = jnp.min(jnp.where(d2s[t] == m, li, float(_CW)),
                              axis=1, keepdims=True)
                sv_ref[:, g * _K + k:g * _K + k + 1] = m
                si_ref[:, g * _K + k:g * _K + k + 1] = aml + float(g * _CW)
                d2s[t] = jnp.where(li == aml, jnp.inf, d2s[t])
    vals = sv_ref[...]
    idxs = si_ref[...]
    for k in range(_K):
        m = jnp.min(vals, axis=1, keepdims=True)
        j = jnp.min(jnp.where(vals == m, idxs, float(_NPAD)),
                    axis=1, keepdims=True)
        nbr_ref[:, k:k + 1] = j.astype(jnp.int32)
        sel = jnp.logical_and(vals == m, idxs == j)
        vals = jnp.where(sel, jnp.inf, vals)


_E = _K * _M
_NW = 32
_EPW = _E // _NW
_GCH = 200


def _make_sc_gather():
    mesh = plsc.VectorSubcoreMesh(core_axis_name="c", subcore_axis_name="s")

    @functools.partial(
        pl.kernel, mesh=mesh,
        out_type=jax.ShapeDtypeStruct((_E, _D), jnp.float32),
        scratch_types=[
            pltpu.VMEM((_EPW,), jnp.int32),
            pltpu.VMEM((_GCH, _D), jnp.float32),
            pltpu.SemaphoreType.DMA,
        ],
    )
    def sc_gather(table_hbm, idx_hbm, out_hbm, idx_v, buf, sem):
        wid = lax.axis_index("s") * 2 + lax.axis_index("c")
        base = wid * _EPW
        pltpu.sync_copy(idx_hbm.at[pl.ds(base, _EPW)], idx_v)

        def step(ch, _):
            pltpu.async_copy(
                table_hbm.at[idx_v.at[pl.ds(ch * _GCH, _GCH)]],
                buf, sem).wait()
            pltpu.sync_copy(
                buf, out_hbm.at[pl.ds(base + ch * _GCH, _GCH)])
            return 0

        lax.fori_loop(0, _EPW // _GCH, step, 0)

    return sc_gather


def _u_body(x_ref, p_ref, w1a_ref, w1b_ref, b1_ref, u_ref):
    u_ref[...] = (
        jnp.dot(x_ref[...], w1a_ref[...], preferred_element_type=jnp.float32)
        + jnp.dot(p_ref[...], w1b_ref[...], preferred_element_type=jnp.float32)
        + b1_ref[...]
    )


def _mlp_body(g_ref, ps_ref, w1b_ref, w2_ref, b2_ref, out_ref):
    v = jnp.dot(ps_ref[...], w1b_ref[...], preferred_element_type=jnp.float32)
    acc = jnp.full((_BC, _D), -jnp.inf, dtype=jnp.float32)
    w2 = w2_ref[...]
    b2 = b2_ref[...]
    for k in range(_K):
        h1 = jnp.maximum(g_ref[k] - v, 0.0)
        h2 = jnp.dot(h1, w2, preferred_element_type=jnp.float32) + b2
        acc = jnp.maximum(acc, h2)
    out_ref[...] = jnp.maximum(acc, 0.0)


def kernel(x, pos, batch, W1, b1, W2, b2):
    f32 = jnp.float32
    posp = jnp.pad(pos.astype(f32), ((0, _NPAD - _N), (0, 0)))
    px = posp[:, 0].reshape(_SUB, _LANES)
    py = posp[:, 1].reshape(_SUB, _LANES)
    pz = posp[:, 2].reshape(_SUB, _LANES)

    idx_buf = pl.pallas_call(
        _fps_body,
        in_specs=[
            pl.BlockSpec((_NPAD, 3), lambda: (0, 0)),
            pl.BlockSpec((_SUB, _LANES), lambda: (0, 0)),
            pl.BlockSpec((_SUB, _LANES), lambda: (0, 0)),
            pl.BlockSpec((_SUB, _LANES), lambda: (0, 0)),
        ],
        out_specs=pl.BlockSpec(memory_space=pltpu.SMEM),
        out_shape=jax.ShapeDtypeStruct((_SUB * _IDXL,), jnp.int32),
    )(posp, px, py, pz)
    idx = idx_buf[:_M]

    pos_s = jnp.take(pos, idx, axis=0)

    posq = jnp.pad(pos.astype(f32), ((0, _NPAD - _N), (0, 0)),
                   constant_values=1e4)
    pxr = posq[:, 0].reshape(1, _NPAD)
    pyr = posq[:, 1].reshape(1, _NPAD)
    pzr = posq[:, 2].reshape(1, _NPAD)
    nbr = pl.pallas_call(
        _knn_body,
        grid=(_M // _SUB,),
        in_specs=[
            pl.BlockSpec((_SUB, 3), lambda b: (b, 0)),
            pl.BlockSpec((1, _NPAD), lambda b: (0, 0)),
            pl.BlockSpec((1, _NPAD), lambda b: (0, 0)),
            pl.BlockSpec((1, _NPAD), lambda b: (0, 0)),
        ],
        out_specs=pl.BlockSpec((_SUB, _K), lambda b: (b, 0)),
        out_shape=jax.ShapeDtypeStruct((_M, _K), jnp.int32),
        scratch_shapes=[
            pltpu.VMEM((_SUB, _NCH * _K), f32),
            pltpu.VMEM((_SUB, _NCH * _K), f32),
        ],
    )(pos_s, pxr, pyr, pzr)

    W1a = W1[:_D, :]
    W1b = W1[_D:, :]
    b1r = b1.reshape(1, _D)
    xp = jnp.pad(x.astype(f32), ((0, _NPAD - _N), (0, 0)))
    u = pl.pallas_call(
        _u_body,
        grid=(_NPAD // _RB,),
        in_specs=[
            pl.BlockSpec((_RB, _D), lambda b: (b, 0)),
            pl.BlockSpec((_RB, 3), lambda b: (b, 0)),
            pl.BlockSpec((_D, _D), lambda b: (0, 0)),
            pl.BlockSpec((3, _D), lambda b: (0, 0)),
            pl.BlockSpec((1, _D), lambda b: (0, 0)),
        ],
        out_specs=pl.BlockSpec((_RB, _D), lambda b: (b, 0)),
        out_shape=jax.ShapeDtypeStruct((_NPAD, _D), f32),
    )(xp, posp, W1a, W1b, b1r)

    col = nbr.T.reshape(-1)
    g = _make_sc_gather()(u, col).reshape(_K, _M, _D)

    b2r = b2.reshape(1, _D)
    out = pl.pallas_call(
        _mlp_body,
        grid=(_M // _BC,),
        in_specs=[
            pl.BlockSpec((_K, _BC, _D), lambda b: (0, b, 0)),
            pl.BlockSpec((_BC, 3), lambda b: (b, 0)),
            pl.BlockSpec((3, _D), lambda b: (0, 0)),
            pl.BlockSpec((_D, _D), lambda b: (0, 0)),
            pl.BlockSpec((1, _D), lambda b: (0, 0)),
        ],
        out_specs=pl.BlockSpec((_BC, _D), lambda b: (b, 0)),
        out_shape=jax.ShapeDtypeStruct((_M, _D), f32),
    )(g, pos_s, W1b, W2, b2r)

    return (out, pos_s, jnp.take(batch, idx, axis=0))


# speedup vs baseline: 4.1566x; 1.0070x over previous
import functools

import jax
import jax.numpy as jnp
from jax import lax
from jax.experimental import pallas as pl
from jax.experimental.pallas import tpu as pltpu
from jax.experimental.pallas import tpu_sc as plsc

_N = 10000
_M = 5000
_K = 32
_D = 128
_SUB = 8
_LANES = 1280
_NPAD = _SUB * _LANES
_IDXL = 640
_BC = 200
_RB = 512


def _fps_body(pr_ref, px_ref, py_ref, pz_ref, idx_ref):
    si = jax.lax.broadcasted_iota(jnp.int32, (_SUB, _LANES), 0)
    li = jax.lax.broadcasted_iota(jnp.int32, (_SUB, _LANES), 1)
    fj = si * _LANES + li
    d0 = jnp.where(fj < _N, jnp.inf, -jnp.inf)

    def body(i, carry):
        dists, cur = carry
        idx_ref[i] = cur
        c = pr_ref[pl.ds(cur, 1), :]
        dx = px_ref[...] - c[0:1, 0:1]
        d = dx * dx
        dy = py_ref[...] - c[0:1, 1:2]
        d = d + dy * dy
        dz = pz_ref[...] - c[0:1, 2:3]
        d = d + dz * dz
        nd = jnp.minimum(dists, d)
        rm = jnp.max(nd, axis=1, keepdims=True)
        la = jnp.min(jnp.where(nd == rm,
                               jax.lax.broadcasted_iota(
                                   jnp.int32, (_SUB, _LANES), 1),
                               _LANES), axis=1, keepdims=True)
        roff = jax.lax.broadcasted_iota(jnp.int32, (_SUB, 1), 0) * _LANES
        ra = la + roff
        gm = jnp.max(rm)
        nxt = jnp.min(jnp.where(rm == gm, ra, _NPAD)).astype(jnp.int32)
        return nd, nxt

    jax.lax.fori_loop(0, _M, body, (d0, jnp.int32(0)))


_NCH = 16
_CW = _NPAD // _NCH


_CG = 4


def _knn_body(ps_ref, px_ref, py_ref, pz_ref, nbr_ref, sv_ref, si_ref):
    ps = ps_ref[...]
    cx = ps[:, 0:1]
    cy = ps[:, 1:2]
    cz = ps[:, 2:3]
    li = jax.lax.broadcasted_iota(jnp.int32, (_SUB, _CW), 1).astype(jnp.float32)
    for grp in range(_NCH // _CG):
        d2s = []
        for t in range(_CG):
            g = grp * _CG + t
            sl = slice(g * _CW, (g + 1) * _CW)
            dx = cx - px_ref[:, sl]
            dy = cy - py_ref[:, sl]
            dz = cz - pz_ref[:, sl]
            d2s.append(dx * dx + dy * dy + dz * dz)
        for k in range(_K):
            for t in range(_CG):
                g = grp * _CG + t
                m = jnp.min(d2s[t], axis=1, keepdims=True)
                aml = jnp.min(jnp.where(d2s[t] == m, li, float(_CW)),
                              axis=1, keepdims=True)
                sv_ref[:, g * _K + k:g * _K + k + 1] = m
                si_ref[:, g * _K + k:g * _K + k + 1] = aml + float(g * _CW)
                d2s[t] = jnp.where(li == aml, jnp.inf, d2s[t])
    vals = sv_ref[...]
    idxs = si_ref[...]
    for k in range(_K):
        m = jnp.min(vals, axis=1, keepdims=True)
        j = jnp.min(jnp.where(vals == m, idxs, float(_NPAD)),
                    axis=1, keepdims=True)
        nbr_ref[:, k:k + 1] = j.astype(jnp.int32)
        sel = jnp.logical_and(vals == m, idxs == j)
        vals = jnp.where(sel, jnp.inf, vals)


_E = _K * _M
_NW = 32
_EPW = _E // _NW
_GCH = 200


def _make_sc_gather():
    mesh = plsc.VectorSubcoreMesh(core_axis_name="c", subcore_axis_name="s")

    @functools.partial(
        pl.kernel, mesh=mesh,
        out_type=jax.ShapeDtypeStruct((_E, _D), jnp.float32),
        scratch_types=[
            pltpu.VMEM((_EPW,), jnp.int32),
            pltpu.VMEM((_GCH, _D), jnp.float32),
            pltpu.SemaphoreType.DMA,
        ],
    )
    def sc_gather(table_hbm, idx_hbm, out_hbm, idx_v, buf, sem):
        wid = lax.axis_index("s") * 2 + lax.axis_index("c")
        base = wid * _EPW
        pltpu.sync_copy(idx_hbm.at[pl.ds(base, _EPW)], idx_v)

        def step(ch, _):
            pltpu.async_copy(
                table_hbm.at[idx_v.at[pl.ds(ch * _GCH, _GCH)]],
                buf, sem).wait()
            pltpu.sync_copy(
                buf, out_hbm.at[pl.ds(base + ch * _GCH, _GCH)])
            return 0

        lax.fori_loop(0, _EPW // _GCH, step, 0)

    return sc_gather


def _u_body(x_ref, p_ref, w1a_ref, w1b_ref, b1_ref, u_ref):
    u_ref[...] = (
        jnp.dot(x_ref[...], w1a_ref[...], preferred_element_type=jnp.float32)
        + jnp.dot(p_ref[...], w1b_ref[...], preferred_element_type=jnp.float32)
        + b1_ref[...]
    )


def _mlp_body(g_ref, ps_ref, w1b_ref, w2_ref, b2_ref, out_ref):
    v = jnp.dot(ps_ref[...], w1b_ref[...], preferred_element_type=jnp.float32)
    acc = jnp.full((_BC, _D), -jnp.inf, dtype=jnp.float32)
    w2 = w2_ref[...]
    b2 = b2_ref[...]
    for k in range(_K):
        h1 = jnp.maximum(g_ref[k] - v, 0.0)
        h2 = jnp.dot(h1, w2, preferred_element_type=jnp.float32) + b2
        acc = jnp.maximum(acc, h2)
    out_ref[...] = jnp.maximum(acc, 0.0)


def kernel(x, pos, batch, W1, b1, W2, b2):
    f32 = jnp.float32
    posp = jnp.pad(pos.astype(f32), ((0, _NPAD - _N), (0, 0)))
    px = posp[:, 0].reshape(_SUB, _LANES)
    py = posp[:, 1].reshape(_SUB, _LANES)
    pz = posp[:, 2].reshape(_SUB, _LANES)

    idx_buf = pl.pallas_call(
        _fps_body,
        in_specs=[
            pl.BlockSpec((_NPAD, 3), lambda: (0, 0)),
            pl.BlockSpec((_SUB, _LANES), lambda: (0, 0)),
            pl.BlockSpec((_SUB, _LANES), lambda: (0, 0)),
            pl.BlockSpec((_SUB, _LANES), lambda: (0, 0)),
        ],
        out_specs=pl.BlockSpec(memory_space=pltpu.SMEM),
        out_shape=jax.ShapeDtypeStruct((_SUB * _IDXL,), jnp.int32),
    )(posp, px, py, pz)
    idx = idx_buf[:_M]

    pos_s = jnp.take(pos, idx, axis=0)

    posq = jnp.pad(pos.astype(f32), ((0, _NPAD - _N), (0, 0)),
                   constant_values=1e4)
    pxr = posq[:, 0].reshape(1, _NPAD)
    pyr = posq[:, 1].reshape(1, _NPAD)
    pzr = posq[:, 2].reshape(1, _NPAD)
    nbr = pl.pallas_call(
        _knn_body,
        grid=(_M // _SUB,),
        in_specs=[
            pl.BlockSpec((_SUB, 3), lambda b: (b, 0)),
            pl.BlockSpec((1, _NPAD), lambda b: (0, 0)),
            pl.BlockSpec((1, _NPAD), lambda b: (0, 0)),
            pl.BlockSpec((1, _NPAD), lambda b: (0, 0)),
        ],
        out_specs=pl.BlockSpec((_SUB, _K), lambda b: (b, 0)),
        out_shape=jax.ShapeDtypeStruct((_M, _K), jnp.int32),
        scratch_shapes=[
            pltpu.VMEM((_SUB, _NCH * _K), f32),
            pltpu.VMEM((_SUB, _NCH * _K), f32),
        ],
    )(pos_s, pxr, pyr, pzr)

    W1a = W1[:_D, :]
    W1b = W1[_D:, :]
    b1r = b1.reshape(1, _D)
    xp = jnp.pad(x.astype(f32), ((0, _NPAD - _N), (0, 0)))
    u = pl.pallas_call(
        _u_body,
        grid=(_NPAD // _RB,),
        in_specs=[
            pl.BlockSpec((_RB, _D), lambda b: (b, 0)),
            pl.BlockSpec((_RB, 3), lambda b: (b, 0)),
            pl.BlockSpec((_D, _D), lambda b: (0, 0)),
            pl.BlockSpec((3, _D), lambda b: (0, 0)),
            pl.BlockSpec((1, _D), lambda b: (0, 0)),
        ],
        out_specs=pl.BlockSpec((_RB, _D), lambda b: (b, 0)),
        out_shape=jax.ShapeDtypeStruct((_NPAD, _D), f32),
    )(xp, posp, W1a, W1b, b1r)

    col = nbr.T.reshape(-1)
    g = _make_sc_gather()(u, col).reshape(_K, _M, _D)

    b2r = b2.reshape(1, _D)
    out = pl.pallas_call(
        _mlp_body,
        grid=(_M // _BC,),
        in_specs=[
            pl.BlockSpec((_K, _BC, _D), lambda b: (0, b, 0)),
            pl.BlockSpec((_BC, 3), lambda b: (b, 0)),
            pl.BlockSpec((3, _D), lambda b: (0, 0)),
            pl.BlockSpec((_D, _D), lambda b: (0, 0)),
            pl.BlockSpec((1, _D), lambda b: (0, 0)),
        ],
        out_specs=pl.BlockSpec((_BC, _D), lambda b: (b, 0)),
        out_shape=jax.ShapeDtypeStruct((_M, _D), f32),
    )(g, pos_s, W1b, W2, b2r)

    return (out, pos_s, jnp.take(batch, idx, axis=0))
